# baseline (speedup 1.0000x reference)
import jax
import jax.numpy as jnp
from jax import lax
from jax.experimental import pallas as pl
from jax.experimental.pallas import tpu as pltpu
from jax.experimental.pallas import tpu_sc as plsc

B, C, H, W, K = 16, 80, 128, 128, 128
HW = H * W
HM_ELEMS = B * C * H * W
LANES = 128
ROWS = HM_ELEMS // LANES
BLOCK_ROWS = 8192
GRID = ROWS // BLOCK_ROWS

NC, NS, L = 2, 16, 16
NW = NC * NS
K_PER_W = (B * K) // NW
VREGS = K_PER_W // L

HM_WEIGHT = 1.0
WH_WEIGHT = 0.1


def _mse_body(a_ref, b_ref, out_ref, acc_ref):
    i = pl.program_id(0)

    @pl.when(i == 0)
    def _init():
        acc_ref[...] = jnp.zeros_like(acc_ref)

    d = a_ref[...] - b_ref[...]
    acc_ref[...] += jnp.sum((d * d).reshape(BLOCK_ROWS // 8, 8, LANES), axis=0)

    @pl.when(i == GRID - 1)
    def _fin():
        out_ref[0, 0] = jnp.sum(acc_ref[...])


def _sc_body(wh_hbm, ind_hbm, msk_hbm, tgt_hbm, out_hbm, idx_v, msk_v,
             addr_v, pred_v, tgt_v, res_v, sem):
    wid = lax.axis_index("s") * NC + lax.axis_index("c")
    g0 = wid * K_PER_W

    pltpu.sync_copy(ind_hbm.at[pl.ds(g0, K_PER_W)], idx_v)
    pltpu.sync_copy(msk_hbm.at[pl.ds(g0, K_PER_W)], msk_v)
    pltpu.sync_copy(tgt_hbm.at[pl.ds(2 * g0, 2 * K_PER_W)], tgt_v)

    for i in range(VREGS):
        lane_g = g0 + i * L + lax.iota(jnp.int32, L)
        base = (lane_g >> 7) * (2 * HW)
        idx = idx_v[pl.ds(i * L, L)]
        addr_v[pl.ds(i * L, L)] = base + idx
        addr_v[pl.ds(K_PER_W + i * L, L)] = base + HW + idx

    pltpu.async_copy(wh_hbm.at[addr_v], pred_v, sem).wait()

    acc = jnp.zeros((L,), jnp.float32)
    cnt = jnp.zeros((L,), jnp.float32)
    for i in range(VREGS):
        mskf = msk_v[pl.ds(i * L, L)].astype(jnp.float32)
        px = pred_v[pl.ds(i * L, L)]
        py = pred_v[pl.ds(K_PER_W + i * L, L)]
        tbase = lax.iota(jnp.int32, L) * 2 + i * 2 * L
        tx = plsc.load_gather(tgt_v, [tbase])
        ty = plsc.load_gather(tgt_v, [tbase + 1])
        acc = acc + mskf * (jnp.abs(px - tx) + jnp.abs(py - ty))
        cnt = cnt + mskf

    res_v[0, :] = acc
    res_v[1, :] = cnt
    pltpu.sync_copy(res_v, out_hbm.at[wid])


def _combine_body(mse_ref, sc_ref, loss_ref, hm_ref, wh_ref):
    hm = mse_ref[0, 0] / HM_ELEMS
    part = sc_ref[...]
    l1 = jnp.sum(part[:, 0, :])
    cnt = jnp.sum(part[:, 1, :]) * 2.0
    wh = l1 / (cnt + 0.0001)
    hm_ref[0, 0] = hm
    wh_ref[0, 0] = wh
    loss_ref[0, 0] = HM_WEIGHT * hm + WH_WEIGHT * wh


def kernel(in_hm, out_hm, in_wh, out_wh, reg_mask, ind):
    a = in_hm.reshape(ROWS, LANES)
    b = out_hm.reshape(ROWS, LANES)

    mse_sum = pl.pallas_call(
        _mse_body,
        grid=(GRID,),
        in_specs=[
            pl.BlockSpec((BLOCK_ROWS, LANES), lambda i: (i, 0)),
            pl.BlockSpec((BLOCK_ROWS, LANES), lambda i: (i, 0)),
        ],
        out_specs=pl.BlockSpec(memory_space=pltpu.SMEM),
        out_shape=jax.ShapeDtypeStruct((1, 1), jnp.float32),
        scratch_shapes=[pltpu.VMEM((8, LANES), jnp.float32)],
        compiler_params=pltpu.CompilerParams(
            dimension_semantics=("arbitrary",)),
    )(a, b)

    sc_gather = pl.kernel(
        _sc_body,
        out_type=jax.ShapeDtypeStruct((NW, 2, L), jnp.float32),
        mesh=plsc.VectorSubcoreMesh(
            core_axis_name="c", subcore_axis_name="s", num_cores=NC),
        scratch_types=[
            pltpu.VMEM((K_PER_W,), jnp.int32),
            pltpu.VMEM((K_PER_W,), jnp.int32),
            pltpu.VMEM((2 * K_PER_W,), jnp.int32),
            pltpu.VMEM((2 * K_PER_W,), jnp.float32),
            pltpu.VMEM((2 * K_PER_W,), jnp.float32),
            pltpu.VMEM((2, L), jnp.float32),
            pltpu.SemaphoreType.DMA,
        ],
        compiler_params=pltpu.CompilerParams(needs_layout_passes=False),
    )
    sc_part = sc_gather(
        out_wh.reshape(B * 2 * HW),
        ind.reshape(B * K),
        reg_mask.reshape(B * K),
        in_wh.reshape(B * 2 * K),
    )

    hm_loss = mse_sum[0, 0] / HM_ELEMS
    l1 = jnp.sum(sc_part[:, 0, :])
    cnt = jnp.sum(sc_part[:, 1, :]) * 2.0
    wh_loss = l1 / (cnt + 0.0001)
    loss = HM_WEIGHT * hm_loss + WH_WEIGHT * wh_loss
    return (loss, hm_loss, wh_loss)


# speedup vs baseline: 1.0634x; 1.0634x over previous
import jax
import jax.numpy as jnp
from jax import lax
from jax.experimental import pallas as pl
from jax.experimental.pallas import tpu as pltpu
from jax.experimental.pallas import tpu_sc as plsc

B, C, H, W, K = 16, 80, 128, 128, 128
HW = H * W
HM_ELEMS = B * C * H * W
LANES = 128
ROWS = HM_ELEMS // LANES
BLOCK_ROWS = 8192
GRID = ROWS // BLOCK_ROWS

NC, NS, L = 2, 16, 16
NW = NC * NS
K_PER_W = (B * K) // NW
VREGS = K_PER_W // L

HM_WEIGHT = 1.0
WH_WEIGHT = 0.1


def _mse_body(a_ref, b_ref, out_ref, acc_ref):
    i = pl.program_id(0)

    @pl.when(i == 0)
    def _init():
        acc_ref[...] = jnp.zeros_like(acc_ref)

    d = a_ref[...] - b_ref[...]
    acc_ref[...] += jnp.sum((d * d).reshape(BLOCK_ROWS // 8, 8, LANES), axis=0)

    @pl.when(i == GRID - 1)
    def _fin():
        out_ref[0, 0] = jnp.sum(acc_ref[...])


def _sc_body(wh_hbm, ind_hbm, msk_hbm, tgt_hbm, out_hbm, idx_v, msk_v,
             addr_v, pred_v, tgt_v, res_v, sem_i, sem_m, sem_t, sem_g):
    wid = lax.axis_index("s") * NC + lax.axis_index("c")
    g0 = wid * K_PER_W

    cp_i = pltpu.async_copy(ind_hbm.at[pl.ds(g0, K_PER_W)], idx_v, sem_i)
    cp_m = pltpu.async_copy(msk_hbm.at[pl.ds(g0, K_PER_W)], msk_v, sem_m)
    cp_t = pltpu.async_copy(
        tgt_hbm.at[pl.ds(2 * g0, 2 * K_PER_W)], tgt_v, sem_t)
    cp_i.wait()

    for i in range(VREGS):
        lane_g = g0 + i * L + lax.iota(jnp.int32, L)
        base = (lane_g >> 7) * (2 * HW)
        idx = idx_v[pl.ds(i * L, L)]
        addr_v[pl.ds(i * L, L)] = base + idx
        addr_v[pl.ds(K_PER_W + i * L, L)] = base + HW + idx

    cp_g = pltpu.async_copy(wh_hbm.at[addr_v], pred_v, sem_g)
    cp_m.wait()
    cp_t.wait()
    cp_g.wait()

    acc = jnp.zeros((L,), jnp.float32)
    cnt = jnp.zeros((L,), jnp.float32)
    for i in range(VREGS):
        mskf = msk_v[pl.ds(i * L, L)].astype(jnp.float32)
        px = pred_v[pl.ds(i * L, L)]
        py = pred_v[pl.ds(K_PER_W + i * L, L)]
        tbase = lax.iota(jnp.int32, L) * 2 + i * 2 * L
        tx = plsc.load_gather(tgt_v, [tbase])
        ty = plsc.load_gather(tgt_v, [tbase + 1])
        acc = acc + mskf * (jnp.abs(px - tx) + jnp.abs(py - ty))
        cnt = cnt + mskf

    res_v[0, :] = acc
    res_v[1, :] = cnt
    pltpu.sync_copy(res_v, out_hbm.at[wid])


def _combine_body(mse_ref, sc_ref, loss_ref, hm_ref, wh_ref):
    hm = mse_ref[0, 0] / HM_ELEMS
    part = sc_ref[...]
    l1 = jnp.sum(part[:, 0, :])
    cnt = jnp.sum(part[:, 1, :]) * 2.0
    wh = l1 / (cnt + 0.0001)
    hm_ref[0, 0] = hm
    wh_ref[0, 0] = wh
    loss_ref[0, 0] = HM_WEIGHT * hm + WH_WEIGHT * wh


def kernel(in_hm, out_hm, in_wh, out_wh, reg_mask, ind):
    a = in_hm.reshape(ROWS, LANES)
    b = out_hm.reshape(ROWS, LANES)

    sc_gather = pl.kernel(
        _sc_body,
        out_type=jax.ShapeDtypeStruct((NW, 2, L), jnp.float32),
        mesh=plsc.VectorSubcoreMesh(
            core_axis_name="c", subcore_axis_name="s", num_cores=NC),
        scratch_types=[
            pltpu.VMEM((K_PER_W,), jnp.int32),
            pltpu.VMEM((K_PER_W,), jnp.int32),
            pltpu.VMEM((2 * K_PER_W,), jnp.int32),
            pltpu.VMEM((2 * K_PER_W,), jnp.float32),
            pltpu.VMEM((2 * K_PER_W,), jnp.float32),
            pltpu.VMEM((2, L), jnp.float32),
            pltpu.SemaphoreType.DMA,
            pltpu.SemaphoreType.DMA,
            pltpu.SemaphoreType.DMA,
            pltpu.SemaphoreType.DMA,
        ],
        compiler_params=pltpu.CompilerParams(needs_layout_passes=False),
    )
    sc_part = sc_gather(
        out_wh.reshape(B * 2 * HW),
        ind.reshape(B * K),
        reg_mask.reshape(B * K),
        in_wh.reshape(B * 2 * K),
    )

    mse_sum = pl.pallas_call(
        _mse_body,
        grid=(GRID,),
        in_specs=[
            pl.BlockSpec((BLOCK_ROWS, LANES), lambda i: (i, 0)),
            pl.BlockSpec((BLOCK_ROWS, LANES), lambda i: (i, 0)),
        ],
        out_specs=pl.BlockSpec(memory_space=pltpu.SMEM),
        out_shape=jax.ShapeDtypeStruct((1, 1), jnp.float32),
        scratch_shapes=[pltpu.VMEM((8, LANES), jnp.float32)],
        compiler_params=pltpu.CompilerParams(
            dimension_semantics=("arbitrary",)),
    )(a, b)

    loss, hm_loss, wh_loss = pl.pallas_call(
        _combine_body,
        in_specs=[
            pl.BlockSpec(memory_space=pltpu.SMEM),
            pl.BlockSpec(memory_space=pltpu.VMEM),
        ],
        out_specs=[
            pl.BlockSpec(memory_space=pltpu.SMEM),
            pl.BlockSpec(memory_space=pltpu.SMEM),
            pl.BlockSpec(memory_space=pltpu.SMEM),
        ],
        out_shape=[
            jax.ShapeDtypeStruct((1, 1), jnp.float32),
            jax.ShapeDtypeStruct((1, 1), jnp.float32),
            jax.ShapeDtypeStruct((1, 1), jnp.float32),
        ],
    )(mse_sum, sc_part)

    return (loss.reshape(()), hm_loss.reshape(()), wh_loss.reshape(()))


# speedup vs baseline: 1.0870x; 1.0221x over previous
import jax
import jax.numpy as jnp
from jax import lax
from jax.experimental import pallas as pl
from jax.experimental.pallas import tpu as pltpu
from jax.experimental.pallas import tpu_sc as plsc

B, C, H, W, K = 16, 80, 128, 128, 128
HW = H * W
HM_ELEMS = B * C * H * W
LANES = 128
ROWS = HM_ELEMS // LANES
BLOCK_ROWS = 8192
GRID = ROWS // BLOCK_ROWS

NC, NS, L = 1, 16, 16
NW = NC * NS
K_PER_W = (B * K) // NW
VREGS = K_PER_W // L

HM_WEIGHT = 1.0
WH_WEIGHT = 0.1


def _mse_body(a_ref, b_ref, out_ref, acc_ref):
    i = pl.program_id(0)

    @pl.when(i == 0)
    def _init():
        acc_ref[...] = jnp.zeros_like(acc_ref)

    d = a_ref[...] - b_ref[...]
    acc_ref[...] += jnp.sum((d * d).reshape(BLOCK_ROWS // 8, 8, LANES), axis=0)

    @pl.when(i == GRID - 1)
    def _fin():
        out_ref[0, 0] = jnp.sum(acc_ref[...])


def _sc_body(wh_hbm, ind_hbm, msk_hbm, tgt_hbm, out_hbm, idx_v, msk_v,
             addrx_v, addry_v, predx_v, predy_v, tgt_v, res_v,
             sem_i, sem_m, sem_t, sem_gx, sem_gy):
    wid = lax.axis_index("s") * NC + lax.axis_index("c")
    g0 = wid * K_PER_W

    cp_i = pltpu.async_copy(ind_hbm.at[pl.ds(g0, K_PER_W)], idx_v, sem_i)
    cp_m = pltpu.async_copy(msk_hbm.at[pl.ds(g0, K_PER_W)], msk_v, sem_m)
    cp_t = pltpu.async_copy(
        tgt_hbm.at[pl.ds(2 * g0, 2 * K_PER_W)], tgt_v, sem_t)
    cp_i.wait()

    for i in range(VREGS):
        lane_g = g0 + i * L + lax.iota(jnp.int32, L)
        base = (lane_g >> 7) * (2 * HW)
        idx = idx_v[pl.ds(i * L, L)]
        addrx_v[pl.ds(i * L, L)] = base + idx
        addry_v[pl.ds(i * L, L)] = base + HW + idx

    cp_gx = pltpu.async_copy(wh_hbm.at[addrx_v], predx_v, sem_gx)
    cp_gy = pltpu.async_copy(wh_hbm.at[addry_v], predy_v, sem_gy)
    cp_m.wait()
    cp_t.wait()
    cp_gx.wait()
    cp_gy.wait()

    acc = jnp.zeros((L,), jnp.float32)
    cnt = jnp.zeros((L,), jnp.float32)
    for i in range(VREGS):
        mskf = msk_v[pl.ds(i * L, L)].astype(jnp.float32)
        px = predx_v[pl.ds(i * L, L)]
        py = predy_v[pl.ds(i * L, L)]
        tbase = lax.iota(jnp.int32, L) * 2 + i * 2 * L
        tx = plsc.load_gather(tgt_v, [tbase])
        ty = plsc.load_gather(tgt_v, [tbase + 1])
        acc = acc + mskf * (jnp.abs(px - tx) + jnp.abs(py - ty))
        cnt = cnt + mskf

    res_v[0, :] = acc
    res_v[1, :] = cnt
    pltpu.sync_copy(res_v, out_hbm.at[wid])


def _combine_body(mse_ref, sc_ref, loss_ref, hm_ref, wh_ref):
    hm = mse_ref[0, 0] / HM_ELEMS
    part = sc_ref[...]
    l1 = jnp.sum(part[:, 0, :])
    cnt = jnp.sum(part[:, 1, :]) * 2.0
    wh = l1 / (cnt + 0.0001)
    hm_ref[0, 0] = hm
    wh_ref[0, 0] = wh
    loss_ref[0, 0] = HM_WEIGHT * hm + WH_WEIGHT * wh


def kernel(in_hm, out_hm, in_wh, out_wh, reg_mask, ind):
    a = in_hm.reshape(ROWS, LANES)
    b = out_hm.reshape(ROWS, LANES)

    sc_gather = pl.kernel(
        _sc_body,
        out_type=jax.ShapeDtypeStruct((NW, 2, L), jnp.float32),
        mesh=plsc.VectorSubcoreMesh(
            core_axis_name="c", subcore_axis_name="s", num_cores=NC),
        scratch_types=[
            pltpu.VMEM((K_PER_W,), jnp.int32),
            pltpu.VMEM((K_PER_W,), jnp.int32),
            pltpu.VMEM((K_PER_W,), jnp.int32),
            pltpu.VMEM((K_PER_W,), jnp.int32),
            pltpu.VMEM((K_PER_W,), jnp.float32),
            pltpu.VMEM((K_PER_W,), jnp.float32),
            pltpu.VMEM((2 * K_PER_W,), jnp.float32),
            pltpu.VMEM((2, L), jnp.float32),
            pltpu.SemaphoreType.DMA,
            pltpu.SemaphoreType.DMA,
            pltpu.SemaphoreType.DMA,
            pltpu.SemaphoreType.DMA,
            pltpu.SemaphoreType.DMA,
        ],
        compiler_params=pltpu.CompilerParams(needs_layout_passes=False),
    )
    sc_part = sc_gather(
        out_wh.reshape(B * 2 * HW),
        ind.reshape(B * K),
        reg_mask.reshape(B * K),
        in_wh.reshape(B * 2 * K),
    )

    mse_sum = pl.pallas_call(
        _mse_body,
        grid=(GRID,),
        in_specs=[
            pl.BlockSpec((BLOCK_ROWS, LANES), lambda i: (i, 0)),
            pl.BlockSpec((BLOCK_ROWS, LANES), lambda i: (i, 0)),
        ],
        out_specs=pl.BlockSpec(memory_space=pltpu.SMEM),
        out_shape=jax.ShapeDtypeStruct((1, 1), jnp.float32),
        scratch_shapes=[pltpu.VMEM((8, LANES), jnp.float32)],
        compiler_params=pltpu.CompilerParams(
            dimension_semantics=("arbitrary",)),
    )(a, b)

    loss, hm_loss, wh_loss = pl.pallas_call(
        _combine_body,
        in_specs=[
            pl.BlockSpec(memory_space=pltpu.SMEM),
            pl.BlockSpec(memory_space=pltpu.VMEM),
        ],
        out_specs=[
            pl.BlockSpec(memory_space=pltpu.SMEM),
            pl.BlockSpec(memory_space=pltpu.SMEM),
            pl.BlockSpec(memory_space=pltpu.SMEM),
        ],
        out_shape=[
            jax.ShapeDtypeStruct((1, 1), jnp.float32),
            jax.ShapeDtypeStruct((1, 1), jnp.float32),
            jax.ShapeDtypeStruct((1, 1), jnp.float32),
        ],
    )(mse_sum, sc_part)

    return (loss.reshape(()), hm_loss.reshape(()), wh_loss.reshape(()))
